# baseline (speedup 1.0000x reference)
import functools

import jax
import jax.numpy as jnp
from jax import lax
from jax.experimental import pallas as pl
from jax.experimental.pallas import tpu as pltpu
from jax.experimental.pallas import tpu_sc as plsc

_NC = 2
_NS = 16
_NW = _NC * _NS
_CH = 80

_MESH = plsc.VectorSubcoreMesh(
    core_axis_name="c", subcore_axis_name="s", num_cores=_NC, num_subcores=_NS)


def _preA_call(X, Wl2, bl2, Wr2, br2):
    N, G = X.shape
    H = Wl2.shape[1]
    f32 = jnp.float32

    def body(X_ref, Wl_ref, bl_ref, Wr_ref, br_ref, xl_ref, xr_ref):
        hp = jax.lax.Precision.HIGHEST
        X = X_ref[...]
        xl_ref[...] = jax.lax.dot_general(
            X, Wl_ref[...], (((1,), (0,)), ((), ())), precision=hp,
            preferred_element_type=f32) + bl_ref[...]
        xr_ref[...] = jax.lax.dot_general(
            X, Wr_ref[...], (((1,), (0,)), ((), ())), precision=hp,
            preferred_element_type=f32) + br_ref[...]

    return pl.pallas_call(
        body,
        out_shape=(
            jax.ShapeDtypeStruct((N, H), f32),
            jax.ShapeDtypeStruct((N, H), f32),
        ),
    )(X, Wl2, bl2, Wr2, br2)


def _preB_call(X, Mu, Var, W, S):
    N, G = X.shape
    K = Mu.shape[0]
    f32 = jnp.float32
    NB = 5
    BR = N // NB

    def body(X_ref, Mu_ref, Var_ref, W_ref, S_ref,
             P_ref, Ppad_ref, ll_ref):
        i = pl.program_id(0)
        Wv = W_ref[...]
        Wm = jnp.max(Wv, axis=1, keepdims=True)
        Pe = jnp.exp(Wv - Wm)
        P = Pe / jnp.sum(Pe, axis=1, keepdims=True)
        P_ref[...] = P
        Ppad_ref[...] = jnp.concatenate(
            [P, jnp.ones((BR, 1), f32), jnp.zeros((BR, 1), f32)], axis=1)

        X = X_ref[...]
        S = S_ref[...]
        iv = 1.0 / Var_ref[...]
        Miv = Mu_ref[...] * iv
        dn = (((1,), (1,)), ((), ()))
        hp = jax.lax.Precision.HIGHEST
        q0 = jax.lax.dot_general(X * X, iv, dn, precision=hp,
                                 preferred_element_type=f32)
        q1 = jax.lax.dot_general(X, Miv, dn, precision=hp,
                                 preferred_element_type=f32)
        c2 = jnp.sum(Mu_ref[...] * Miv, axis=1)
        F = -0.5 * (q0 - 2.0 * S * q1 + (S * S) * c2[None, :])
        part = jnp.reshape(jnp.sum(P * F) / N, (1, 1))

        @pl.when(i == 0)
        def _init():
            ll_ref[...] = jnp.zeros((1, 1), f32)

        ll_ref[...] += part

    row = lambda i: (i, 0)
    fix = lambda i: (0, 0)
    return pl.pallas_call(
        body,
        grid=(NB,),
        in_specs=[
            pl.BlockSpec((BR, G), row),
            pl.BlockSpec((K, G), fix),
            pl.BlockSpec((K, G), fix),
            pl.BlockSpec((BR, K), row),
            pl.BlockSpec((BR, 1), row),
        ],
        out_specs=(
            pl.BlockSpec((BR, K), row),
            pl.BlockSpec((BR, K + 2), row),
            pl.BlockSpec((1, 1), fix),
        ),
        out_shape=(
            jax.ShapeDtypeStruct((N, K), f32),
            jax.ShapeDtypeStruct((N, K + 2), f32),
            jax.ShapeDtypeStruct((1, 1), f32),
        ),
    )(X, Mu, Var, W, S)


def _phase1_call(xla, xlb, xra, xrb, src1, dst1, att16):
    H = 8
    H2 = H // 2
    NH2 = xla.shape[0]
    E = src1.shape[0]
    EPP = E // _NW
    NG = EPP // 16
    f32, i32 = jnp.float32, jnp.int32

    @functools.partial(
        pl.kernel,
        out_type=(
            jax.ShapeDtypeStruct((E,), f32),
            jax.ShapeDtypeStruct((_NW * 16,), f32),
        ),
        mesh=_MESH,
        compiler_params=pltpu.CompilerParams(
            use_tc_tiling_on_sc=False, needs_layout_passes=False),
        scratch_types=[
            pltpu.VMEM((NH2,), f32),
            pltpu.VMEM((NH2,), f32),
            pltpu.VMEM((EPP,), i32),
            pltpu.VMEM((EPP,), i32),
            pltpu.VMEM((EPP,), f32),
            pltpu.VMEM((16,), f32),
            pltpu.VMEM((16,), f32),
        ],
    )
    def k(xla_hbm, xlb_hbm, xra_hbm, xrb_hbm, src_hbm, dst_hbm, att_hbm,
          e_out, mx_out, tl_v, tr_v, src_v, dst_v, e_v, att_v, mx_v):
        cid = lax.axis_index("c")
        sid = lax.axis_index("s")
        wid = cid * _NS + sid

        pltpu.sync_copy(att_hbm, att_v)
        pltpu.sync_copy(xla_hbm, tl_v)
        pltpu.sync_copy(xra_hbm, tr_v)
        pltpu.sync_copy(src_hbm.at[pl.ds(wid * EPP, EPP)], src_v)
        pltpu.sync_copy(dst_hbm.at[pl.ds(wid * EPP, EPP)], dst_v)
        att_vec = att_v[...]
        ats = [att_vec[j] for j in range(H)]

        def half(h):
            a = ats[h * H2:(h + 1) * H2]

            def grp(g, mx):
                idxL = src_v[pl.ds(g * 16, 16)] * H2
                idxR = dst_v[pl.ds(g * 16, 16)] * H2
                e16 = None
                for j in range(H2):
                    gl = plsc.load_gather(tl_v, [idxL + j])
                    gr = plsc.load_gather(tr_v, [idxR + j])
                    s = gl + gr
                    m = jnp.where(s > 0, s, jnp.float32(0.2) * s)
                    t = a[j] * m
                    e16 = t if e16 is None else e16 + t
                if h:
                    e16 = e16 + e_v[pl.ds(g * 16, 16)]
                    mx = jnp.maximum(mx, e16)
                e_v[pl.ds(g * 16, 16)] = e16
                return mx

            return grp

        plsc.parallel_loop(0, NG, unroll=2,
                           carry=jnp.full((16,), -3.0e38, f32))(half(0))
        pltpu.sync_copy(xlb_hbm, tl_v)
        pltpu.sync_copy(xrb_hbm, tr_v)
        mx = plsc.parallel_loop(0, NG, unroll=2,
                                carry=jnp.full((16,), -3.0e38, f32))(half(1))

        pltpu.sync_copy(e_v, e_out.at[pl.ds(wid * EPP, EPP)])
        mx_v[...] = mx
        pltpu.sync_copy(mx_v, mx_out.at[pl.ds(wid * 16, 16)])

    return k(xla, xlb, xra, xrb, src1, dst1, att16)


def _phase2_call(Ppad, src1, dstC, e1, mpart, NPAD):
    N, KP = Ppad.shape
    NW, NCH, CH = dstC.shape
    EPP = NCH * CH
    NPT = NPAD // _NS
    GRP = CH // 16
    f32, i32 = jnp.float32, jnp.int32

    @functools.partial(
        pl.kernel,
        out_type=jax.ShapeDtypeStruct((_NC, NPAD, KP), f32),
        mesh=_MESH,
        compiler_params=pltpu.CompilerParams(
            use_tc_tiling_on_sc=False, needs_layout_passes=False),
        scratch_types=[
            pltpu.VMEM((EPP,), i32),
            pltpu.VMEM((NCH, CH), i32),
            pltpu.VMEM((EPP,), f32),
            pltpu.VMEM((CH, KP), f32),
            pltpu.VMEM((CH, KP), f32),
            pltpu.VMEM((NPAD // _NS, KP), f32),
            pltpu.VMEM((NW * 16,), f32),
            pltpu.VMEM_SHARED((NPAD, KP), f32),
            pltpu.SemaphoreType.DMA,
            pltpu.SemaphoreType.DMA,
            pltpu.SemaphoreType.DMA,
            pltpu.SemaphoreType.DMA,
        ],
    )
    def k(p_hbm, src_hbm, dst_hbm, e_hbm, mp_hbm, acc_out,
          src_v, dst_v, ee_v, rows0, rows1, z_v, mp_v, acc_sh,
          gsem0, gsem1, ssem0, ssem1):
        cid = lax.axis_index("c")
        sid = lax.axis_index("s")
        wid = cid * _NS + sid
        rows = (rows0, rows1)
        sems = (gsem0, gsem1)
        ssems = (ssem0, ssem1)

        pltpu.sync_copy(mp_hbm, mp_v)
        gm = mp_v[pl.ds(0, 16)]
        for w in range(1, NW):
            gm = jnp.maximum(gm, mp_v[pl.ds(w * 16, 16)])
        gmax = jnp.max(gm)

        zero16 = jnp.zeros((16,), f32)

        def zr(r, carry):
            for h in range(KP // 16):
                z_v[r, pl.ds(h * 16, 16)] = zero16
            return carry

        lax.fori_loop(0, NPT, zr, 0)
        pltpu.sync_copy(z_v, acc_sh.at[pl.ds(sid * NPT, NPT)])
        plsc.subcore_barrier()

        pltpu.sync_copy(src_hbm.at[pl.ds(wid * EPP, EPP)], src_v)
        pltpu.sync_copy(dst_hbm.at[wid], dst_v)
        pltpu.sync_copy(e_hbm.at[pl.ds(wid * EPP, EPP)], ee_v)
        iota16 = jax.lax.iota(i32, 16)

        def scale_scatter(c, b):
            @plsc.parallel_loop(0, GRP)
            def scale(g):
                e16 = ee_v[pl.ds(c * CH + g * 16, 16)]
                ee16 = jnp.exp(e16 - gmax)
                ridx = g * 16 + iota16
                for h in range(KP // 16):
                    cbase = h * 16
                    for kk in range(16):
                        cidx = jnp.full((16,), cbase + kk, i32)
                        v = plsc.load_gather(rows[b], [ridx, cidx])
                        plsc.store_scatter(rows[b], [ridx, cidx], v * ee16)

            pltpu.async_copy(rows[b], acc_sh.at[dst_v.at[c]], ssems[b],
                             add=True)

        pltpu.async_copy(p_hbm.at[src_v.at[pl.ds(0, CH)]], rows[0], sems[0])

        def body2(i, carry):
            for b in range(2):
                c = i * 2 + b
                nxt = c + 1

                @pl.when(nxt < NCH)
                def _start(nxt=nxt, b=b):
                    @pl.when(nxt >= 2)
                    def _drain(nxt=nxt, b=b):
                        pltpu.make_async_copy(
                            rows[1 - b], acc_sh.at[dst_v.at[nxt - 2]],
                            ssems[1 - b]).wait()

                    pltpu.async_copy(
                        p_hbm.at[src_v.at[pl.ds(nxt * CH, CH)]],
                        rows[1 - b], sems[1 - b])

                pltpu.make_async_copy(
                    p_hbm.at[src_v.at[pl.ds(c * CH, CH)]],
                    rows[b], sems[b]).wait()
                scale_scatter(c, b)
            return carry

        lax.fori_loop(0, NCH // 2, body2, 0)
        if NCH % 2:
            c = NCH - 1
            pltpu.make_async_copy(
                p_hbm.at[src_v.at[pl.ds(c * CH, CH)]], rows[0], sems[0]).wait()
            scale_scatter(c, 0)

        pltpu.make_async_copy(
            rows[(NCH - 2) % 2], acc_sh.at[dst_v.at[NCH - 2]],
            ssems[(NCH - 2) % 2]).wait()
        pltpu.make_async_copy(
            rows[(NCH - 1) % 2], acc_sh.at[dst_v.at[NCH - 1]],
            ssems[(NCH - 1) % 2]).wait()

        plsc.subcore_barrier()
        pltpu.sync_copy(acc_sh.at[pl.ds(sid * NPT, NPT)], z_v)
        pltpu.sync_copy(z_v, acc_out.at[cid, pl.ds(sid * NPT, NPT)])

    return k(Ppad, src1, dstC, e1, mpart)


def _post_call(acc, P, llpart):
    NCd, NPAD, KP = acc.shape
    N, K = P.shape
    f32 = jnp.float32

    def body(acc_ref, P_ref, ll_ref, llo_ref, ceo_ref):
        A = acc_ref[0] + acc_ref[1]
        Bee = A[:N, :K]
        den = A[:N, K:K + 1]
        P = P_ref[...]
        L = jnp.log(P + 1e-8)
        w = jnp.where(den > 0, 1.0 / den, 0.0)
        ceo_ref[...] = jnp.reshape(-jnp.sum(Bee * L * w) / N, (1, 1))
        llo_ref[...] = ll_ref[...]

    return pl.pallas_call(
        body,
        out_shape=(
            jax.ShapeDtypeStruct((1, 1), f32),
            jax.ShapeDtypeStruct((1, 1), f32),
        ),
    )(acc, P, llpart)


def kernel(X, Mu, Var, edge_index, W, S, Wl, bl, Wr, br, att):
    N, G = X.shape
    K = Mu.shape[0]
    E = edge_index.shape[1]
    H = Wl.shape[1]
    EPP = E // _NW
    NCH = EPP // _CH
    assert NCH * _NW * _CH == E and _CH % 16 == 0
    NPAD = ((N + 8 * _NS - 1) // (8 * _NS)) * 8 * _NS

    src1 = edge_index[0]
    dst1 = edge_index[1]
    dstC = edge_index[1].reshape(_NW, NCH, _CH)
    att16 = jnp.zeros((16,), jnp.float32).at[:H].set(att)

    xl, xr = _preA_call(X, Wl, bl.reshape(1, H), Wr, br.reshape(1, H))
    P, Ppad, llpart = _preB_call(X, Mu, Var, W, S)
    H2 = H // 2
    e1, mpart = _phase1_call(
        xl[:, :H2].reshape(-1), xl[:, H2:].reshape(-1),
        xr[:, :H2].reshape(-1), xr[:, H2:].reshape(-1),
        src1, dst1, att16)
    acc = _phase2_call(Ppad, src1, dstC, e1, mpart, NPAD)
    ll, ce = _post_call(acc, P, llpart)
    return (ll[0, 0], ce[0, 0], P)


# speedup vs baseline: 1.9750x; 1.9750x over previous
import functools

import jax
import jax.numpy as jnp
from jax import lax
from jax.experimental import pallas as pl
from jax.experimental.pallas import tpu as pltpu
from jax.experimental.pallas import tpu_sc as plsc

_NC = 2
_NS = 16
_NW = _NC * _NS
_CH = 80

_MESH = plsc.VectorSubcoreMesh(
    core_axis_name="c", subcore_axis_name="s", num_cores=_NC, num_subcores=_NS)


def _preA_call(X, Wl2, bl2, Wr2, br2):
    N, G = X.shape
    H = Wl2.shape[1]
    f32 = jnp.float32

    def body(X_ref, Wl_ref, bl_ref, Wr_ref, br_ref, xl_ref, xr_ref):
        hp = jax.lax.Precision.HIGHEST
        X = X_ref[...]
        xl_ref[...] = jax.lax.dot_general(
            X, Wl_ref[...], (((1,), (0,)), ((), ())), precision=hp,
            preferred_element_type=f32) + bl_ref[...]
        xr_ref[...] = jax.lax.dot_general(
            X, Wr_ref[...], (((1,), (0,)), ((), ())), precision=hp,
            preferred_element_type=f32) + br_ref[...]

    return pl.pallas_call(
        body,
        out_shape=(
            jax.ShapeDtypeStruct((N, H), f32),
            jax.ShapeDtypeStruct((N, H), f32),
        ),
    )(X, Wl2, bl2, Wr2, br2)


def _preB_call(X, Mu, Var, W, S):
    N, G = X.shape
    K = Mu.shape[0]
    f32 = jnp.float32
    NB = 5
    BR = N // NB

    def body(X_ref, Mu_ref, Var_ref, W_ref, S_ref,
             P_ref, Ppad_ref, ll_ref):
        i = pl.program_id(0)
        Wv = W_ref[...]
        Wm = jnp.max(Wv, axis=1, keepdims=True)
        Pe = jnp.exp(Wv - Wm)
        P = Pe / jnp.sum(Pe, axis=1, keepdims=True)
        P_ref[...] = P
        Ppad_ref[...] = jnp.concatenate(
            [P, jnp.ones((BR, 1), f32), jnp.zeros((BR, 1), f32)], axis=1)

        X = X_ref[...]
        S = S_ref[...]
        iv = 1.0 / Var_ref[...]
        Miv = Mu_ref[...] * iv
        dn = (((1,), (1,)), ((), ()))
        hp = jax.lax.Precision.HIGHEST
        q0 = jax.lax.dot_general(X * X, iv, dn, precision=hp,
                                 preferred_element_type=f32)
        q1 = jax.lax.dot_general(X, Miv, dn, precision=hp,
                                 preferred_element_type=f32)
        c2 = jnp.sum(Mu_ref[...] * Miv, axis=1)
        F = -0.5 * (q0 - 2.0 * S * q1 + (S * S) * c2[None, :])
        part = jnp.reshape(jnp.sum(P * F) / N, (1, 1))

        @pl.when(i == 0)
        def _init():
            ll_ref[...] = jnp.zeros((1, 1), f32)

        ll_ref[...] += part

    row = lambda i: (i, 0)
    fix = lambda i: (0, 0)
    return pl.pallas_call(
        body,
        grid=(NB,),
        in_specs=[
            pl.BlockSpec((BR, G), row),
            pl.BlockSpec((K, G), fix),
            pl.BlockSpec((K, G), fix),
            pl.BlockSpec((BR, K), row),
            pl.BlockSpec((BR, 1), row),
        ],
        out_specs=(
            pl.BlockSpec((BR, K), row),
            pl.BlockSpec((BR, K + 2), row),
            pl.BlockSpec((1, 1), fix),
        ),
        out_shape=(
            jax.ShapeDtypeStruct((N, K), f32),
            jax.ShapeDtypeStruct((N, K + 2), f32),
            jax.ShapeDtypeStruct((1, 1), f32),
        ),
    )(X, Mu, Var, W, S)


def _phase1_call(xla, xlb, xra, xrb, src1, dst1, att16):
    H = 8
    H2 = H // 2
    NH2 = xla.shape[0]
    E = src1.shape[0]
    EPP = E // _NW
    NG = EPP // 16
    f32, i32 = jnp.float32, jnp.int32

    @functools.partial(
        pl.kernel,
        out_type=(
            jax.ShapeDtypeStruct((E,), f32),
            jax.ShapeDtypeStruct((_NW * 16,), f32),
        ),
        mesh=_MESH,
        compiler_params=pltpu.CompilerParams(
            use_tc_tiling_on_sc=False, needs_layout_passes=False),
        scratch_types=[
            pltpu.VMEM((NH2,), f32),
            pltpu.VMEM((NH2,), f32),
            pltpu.VMEM((EPP,), i32),
            pltpu.VMEM((EPP,), i32),
            pltpu.VMEM((EPP,), f32),
            pltpu.VMEM((16,), f32),
            pltpu.VMEM((16,), f32),
        ],
    )
    def k(xla_hbm, xlb_hbm, xra_hbm, xrb_hbm, src_hbm, dst_hbm, att_hbm,
          e_out, mx_out, tl_v, tr_v, src_v, dst_v, e_v, att_v, mx_v):
        cid = lax.axis_index("c")
        sid = lax.axis_index("s")
        wid = cid * _NS + sid

        pltpu.sync_copy(att_hbm, att_v)
        pltpu.sync_copy(xla_hbm, tl_v)
        pltpu.sync_copy(xra_hbm, tr_v)
        pltpu.sync_copy(src_hbm.at[pl.ds(wid * EPP, EPP)], src_v)
        pltpu.sync_copy(dst_hbm.at[pl.ds(wid * EPP, EPP)], dst_v)
        att_vec = att_v[...]
        ats = [att_vec[j] for j in range(H)]

        def half(h):
            a = ats[h * H2:(h + 1) * H2]

            def grp(g, mx):
                idxL = src_v[pl.ds(g * 16, 16)] * H2
                idxR = dst_v[pl.ds(g * 16, 16)] * H2
                e16 = None
                for j in range(H2):
                    gl = plsc.load_gather(tl_v, [idxL + j])
                    gr = plsc.load_gather(tr_v, [idxR + j])
                    s = gl + gr
                    m = jnp.where(s > 0, s, jnp.float32(0.2) * s)
                    t = a[j] * m
                    e16 = t if e16 is None else e16 + t
                if h:
                    e16 = e16 + e_v[pl.ds(g * 16, 16)]
                    mx = jnp.maximum(mx, e16)
                e_v[pl.ds(g * 16, 16)] = e16
                return mx

            return grp

        plsc.parallel_loop(0, NG, unroll=2,
                           carry=jnp.full((16,), -3.0e38, f32))(half(0))
        pltpu.sync_copy(xlb_hbm, tl_v)
        pltpu.sync_copy(xrb_hbm, tr_v)
        mx = plsc.parallel_loop(0, NG, unroll=2,
                                carry=jnp.full((16,), -3.0e38, f32))(half(1))

        pltpu.sync_copy(e_v, e_out.at[pl.ds(wid * EPP, EPP)])
        mx_v[...] = mx
        pltpu.sync_copy(mx_v, mx_out.at[pl.ds(wid * 16, 16)])

    return k(xla, xlb, xra, xrb, src1, dst1, att16)


def _phase2_call(Ppad, src1, dstC, e1, mpart, NPAD):
    N, KP = Ppad.shape
    NW, NCH, CH = dstC.shape
    EPP = NCH * CH
    NPT = NPAD // _NS
    GRP = CH // 16
    f32, i32 = jnp.float32, jnp.int32

    @functools.partial(
        pl.kernel,
        out_type=jax.ShapeDtypeStruct((_NC, NPAD, KP), f32),
        mesh=_MESH,
        compiler_params=pltpu.CompilerParams(
            use_tc_tiling_on_sc=False, needs_layout_passes=False),
        scratch_types=[
            pltpu.VMEM((EPP,), i32),
            pltpu.VMEM((NCH, CH), i32),
            pltpu.VMEM((EPP,), f32),
            pltpu.VMEM((CH, KP), f32),
            pltpu.VMEM((CH, KP), f32),
            pltpu.VMEM((NPAD // _NS, KP), f32),
            pltpu.VMEM((NW * 16,), f32),
            pltpu.VMEM_SHARED((NPAD, KP), f32),
            pltpu.SemaphoreType.DMA,
            pltpu.SemaphoreType.DMA,
            pltpu.SemaphoreType.DMA,
            pltpu.SemaphoreType.DMA,
        ],
    )
    def k(p_hbm, src_hbm, dst_hbm, e_hbm, mp_hbm, acc_out,
          src_v, dst_v, ee_v, rows0, rows1, z_v, mp_v, acc_sh,
          gsem0, gsem1, ssem0, ssem1):
        cid = lax.axis_index("c")
        sid = lax.axis_index("s")
        wid = cid * _NS + sid
        rows = (rows0, rows1)
        sems = (gsem0, gsem1)
        ssems = (ssem0, ssem1)

        pltpu.sync_copy(mp_hbm, mp_v)
        gm = mp_v[pl.ds(0, 16)]
        for w in range(1, NW):
            gm = jnp.maximum(gm, mp_v[pl.ds(w * 16, 16)])
        gmax = jnp.max(gm)

        zero16 = jnp.zeros((16,), f32)

        def zr(r, carry):
            for h in range(KP // 16):
                z_v[r, pl.ds(h * 16, 16)] = zero16
            return carry

        lax.fori_loop(0, NPT, zr, 0)
        pltpu.sync_copy(z_v, acc_sh.at[pl.ds(sid * NPT, NPT)])
        plsc.subcore_barrier()

        pltpu.sync_copy(src_hbm.at[pl.ds(wid * EPP, EPP)], src_v)
        pltpu.sync_copy(dst_hbm.at[wid], dst_v)
        pltpu.sync_copy(e_hbm.at[pl.ds(wid * EPP, EPP)], ee_v)
        iota16 = jax.lax.iota(i32, 16)

        def scale_scatter(c, b):
            @plsc.parallel_loop(0, GRP)
            def scale(g):
                e16 = ee_v[pl.ds(c * CH + g * 16, 16)]
                ee16 = jnp.exp(e16 - gmax)
                base = g * 16
                for r in range(16):
                    s = ee16[r]
                    for h in range(KP // 16):
                        rows[b][base + r, pl.ds(h * 16, 16)] = (
                            rows[b][base + r, pl.ds(h * 16, 16)] * s)

            pltpu.async_copy(rows[b], acc_sh.at[dst_v.at[c]], ssems[b],
                             add=True)

        pltpu.async_copy(p_hbm.at[src_v.at[pl.ds(0, CH)]], rows[0], sems[0])

        def body2(i, carry):
            for b in range(2):
                c = i * 2 + b
                nxt = c + 1

                @pl.when(nxt < NCH)
                def _start(nxt=nxt, b=b):
                    @pl.when(nxt >= 2)
                    def _drain(nxt=nxt, b=b):
                        pltpu.make_async_copy(
                            rows[1 - b], acc_sh.at[dst_v.at[nxt - 2]],
                            ssems[1 - b]).wait()

                    pltpu.async_copy(
                        p_hbm.at[src_v.at[pl.ds(nxt * CH, CH)]],
                        rows[1 - b], sems[1 - b])

                pltpu.make_async_copy(
                    p_hbm.at[src_v.at[pl.ds(c * CH, CH)]],
                    rows[b], sems[b]).wait()
                scale_scatter(c, b)
            return carry

        lax.fori_loop(0, NCH // 2, body2, 0)
        if NCH % 2:
            c = NCH - 1
            pltpu.make_async_copy(
                p_hbm.at[src_v.at[pl.ds(c * CH, CH)]], rows[0], sems[0]).wait()
            scale_scatter(c, 0)

        pltpu.make_async_copy(
            rows[(NCH - 2) % 2], acc_sh.at[dst_v.at[NCH - 2]],
            ssems[(NCH - 2) % 2]).wait()
        pltpu.make_async_copy(
            rows[(NCH - 1) % 2], acc_sh.at[dst_v.at[NCH - 1]],
            ssems[(NCH - 1) % 2]).wait()

        plsc.subcore_barrier()
        pltpu.sync_copy(acc_sh.at[pl.ds(sid * NPT, NPT)], z_v)
        pltpu.sync_copy(z_v, acc_out.at[cid, pl.ds(sid * NPT, NPT)])

    return k(Ppad, src1, dstC, e1, mpart)


def _post_call(acc, P, llpart):
    NCd, NPAD, KP = acc.shape
    N, K = P.shape
    f32 = jnp.float32

    def body(acc_ref, P_ref, ll_ref, llo_ref, ceo_ref):
        A = acc_ref[0] + acc_ref[1]
        Bee = A[:N, :K]
        den = A[:N, K:K + 1]
        P = P_ref[...]
        L = jnp.log(P + 1e-8)
        w = jnp.where(den > 0, 1.0 / den, 0.0)
        ceo_ref[...] = jnp.reshape(-jnp.sum(Bee * L * w) / N, (1, 1))
        llo_ref[...] = ll_ref[...]

    return pl.pallas_call(
        body,
        out_shape=(
            jax.ShapeDtypeStruct((1, 1), f32),
            jax.ShapeDtypeStruct((1, 1), f32),
        ),
    )(acc, P, llpart)


def kernel(X, Mu, Var, edge_index, W, S, Wl, bl, Wr, br, att):
    N, G = X.shape
    K = Mu.shape[0]
    E = edge_index.shape[1]
    H = Wl.shape[1]
    EPP = E // _NW
    NCH = EPP // _CH
    assert NCH * _NW * _CH == E and _CH % 16 == 0
    NPAD = ((N + 8 * _NS - 1) // (8 * _NS)) * 8 * _NS

    src1 = edge_index[0]
    dst1 = edge_index[1]
    dstC = edge_index[1].reshape(_NW, NCH, _CH)
    att16 = jnp.zeros((16,), jnp.float32).at[:H].set(att)

    xl, xr = _preA_call(X, Wl, bl.reshape(1, H), Wr, br.reshape(1, H))
    P, Ppad, llpart = _preB_call(X, Mu, Var, W, S)
    H2 = H // 2
    e1, mpart = _phase1_call(
        xl[:, :H2].reshape(-1), xl[:, H2:].reshape(-1),
        xr[:, :H2].reshape(-1), xr[:, H2:].reshape(-1),
        src1, dst1, att16)
    acc = _phase2_call(Ppad, src1, dstC, e1, mpart, NPAD)
    ll, ce = _post_call(acc, P, llpart)
    return (ll[0, 0], ce[0, 0], P)
